# baseline (speedup 1.0000x reference)
import functools

import jax
import jax.numpy as jnp
from jax.experimental import pallas as pl
from jax.experimental.pallas import tpu as pltpu

N_TOK = 32768
N_EMB = 8192
DIM = 256

BN = 512
BK = 1024


def _vq_body(a_ref, b_ref, z_ref, w_ref, idx_ref, bestv_ref, besti_ref):
    j = pl.program_id(1)
    nkt = pl.num_programs(1)

    s = jax.lax.dot_general(
        z_ref[...], w_ref[...],
        dimension_numbers=(((1,), (1,)), ((), ())),
        preferred_element_type=jnp.float32,
    )
    d = (a_ref[...] + b_ref[...]) - 2.0 * s

    lv = jnp.min(d, axis=1, keepdims=True)
    iota = jax.lax.broadcasted_iota(jnp.int32, d.shape, 1)
    li = jnp.min(jnp.where(d == lv, iota, N_EMB), axis=1, keepdims=True)
    li = li + j * BK

    @pl.when(j == 0)
    def _init():
        bestv_ref[...] = lv
        besti_ref[...] = li

    @pl.when(j > 0)
    def _update():
        upd = lv < bestv_ref[...]
        bestv_ref[...] = jnp.where(upd, lv, bestv_ref[...])
        besti_ref[...] = jnp.where(upd, li, besti_ref[...])

    @pl.when(j == nkt - 1)
    def _emit():
        idx_ref[...] = besti_ref[...]


@functools.partial(jax.jit, static_argnames=())
def _vq_indices(z, W, a, b):
    grid = (N_TOK // BN, N_EMB // BK)
    idx = pl.pallas_call(
        _vq_body,
        grid=grid,
        in_specs=[
            pl.BlockSpec((BN, 1), lambda i, j: (i, 0)),
            pl.BlockSpec((1, BK), lambda i, j: (0, j)),
            pl.BlockSpec((BN, DIM), lambda i, j: (i, 0)),
            pl.BlockSpec((BK, DIM), lambda i, j: (j, 0)),
        ],
        out_specs=pl.BlockSpec((BN, 1), lambda i, j: (i, 0)),
        out_shape=jax.ShapeDtypeStruct((N_TOK, 1), jnp.int32),
        scratch_shapes=[
            pltpu.VMEM((BN, 1), jnp.float32),
            pltpu.VMEM((BN, 1), jnp.int32),
        ],
        compiler_params=pltpu.CompilerParams(
            dimension_semantics=("parallel", "arbitrary"),
        ),
    )(a, b, z, W)
    return idx.reshape(N_TOK)


def kernel(z, W):
    a = jnp.sum(z ** 2, axis=1, keepdims=True)
    b = jnp.sum(W ** 2, axis=1).reshape(1, N_EMB)
    indices = _vq_indices(z, W, a, b)
    z_q = z
    return (z_q, indices)


# speedup vs baseline: 1.2043x; 1.2043x over previous
import functools

import jax
import jax.numpy as jnp
from jax.experimental import pallas as pl
from jax.experimental.pallas import tpu as pltpu

N_TOK = 32768
N_EMB = 8192
DIM = 256

BN = 512
BK = 2048


def _vq_body(a_ref, b_ref, z_ref, w_ref, idx_ref, bestv_ref, besti_ref):
    j = pl.program_id(1)
    nkt = pl.num_programs(1)

    w = w_ref[pl.ds(j * BK, BK), :]
    s2 = jax.lax.dot_general(
        z_ref[...], w,
        dimension_numbers=(((1,), (1,)), ((), ())),
        preferred_element_type=jnp.float32,
    )
    d = (a_ref[...] + b_ref[...]) - s2

    lv = jnp.min(d, axis=1, keepdims=True)
    iota = jax.lax.broadcasted_iota(jnp.int32, d.shape, 1)
    li = jnp.min(jnp.where(d == lv, iota, N_EMB), axis=1, keepdims=True)
    li = li + j * BK

    @pl.when(j == 0)
    def _init():
        bestv_ref[...] = lv
        besti_ref[...] = li

    @pl.when(j > 0)
    def _update():
        upd = lv < bestv_ref[...]
        bestv_ref[...] = jnp.where(upd, lv, bestv_ref[...])
        besti_ref[...] = jnp.where(upd, li, besti_ref[...])

    @pl.when(j == nkt - 1)
    def _emit():
        idx_ref[...] = besti_ref[...]


@functools.partial(jax.jit, static_argnames=())
def _vq_indices(z2, W, a, b):
    grid = (N_TOK // BN, N_EMB // BK)
    idx = pl.pallas_call(
        _vq_body,
        grid=grid,
        in_specs=[
            pl.BlockSpec((BN, 1), lambda i, j: (i, 0)),
            pl.BlockSpec((1, BK), lambda i, j: (0, j)),
            pl.BlockSpec((BN, DIM), lambda i, j: (i, 0)),
            pl.BlockSpec((N_EMB, DIM), lambda i, j: (0, 0)),
        ],
        out_specs=pl.BlockSpec((BN, 1), lambda i, j: (i, 0)),
        out_shape=jax.ShapeDtypeStruct((N_TOK, 1), jnp.int32),
        scratch_shapes=[
            pltpu.VMEM((BN, 1), jnp.float32),
            pltpu.VMEM((BN, 1), jnp.int32),
        ],
        compiler_params=pltpu.CompilerParams(
            dimension_semantics=("parallel", "arbitrary"),
        ),
    )(a, b, z2, W)
    return idx.reshape(N_TOK)


def kernel(z, W):
    a = jnp.sum(z ** 2, axis=1, keepdims=True)
    b = jnp.sum(W ** 2, axis=1).reshape(1, N_EMB)
    indices = _vq_indices(z + z, W, a, b)
    z_q = z
    return (z_q, indices)


# speedup vs baseline: 1.5844x; 1.3157x over previous
import functools

import jax
import jax.numpy as jnp
from jax.experimental import pallas as pl
from jax.experimental.pallas import tpu as pltpu

N_TOK = 32768
N_EMB = 8192
DIM = 256

BN = 512
BK = 2048


def _vq_body(a_ref, b_ref, z_ref, w_ref, idx_ref, bestv_ref, besti_ref):
    j = pl.program_id(1)
    nkt = pl.num_programs(1)

    w = w_ref[pl.ds(j * BK, BK), :]
    s2 = jax.lax.dot_general(
        z_ref[...], w,
        dimension_numbers=(((1,), (1,)), ((), ())),
        preferred_element_type=jnp.float32,
    )
    d = (a_ref[...] + b_ref[...]) - s2

    nch = BK // 128
    pv = d[:, 0:128]
    pc = jnp.zeros(pv.shape, jnp.int32)
    for c in range(1, nch):
        dc = d[:, c * 128:(c + 1) * 128]
        m = dc < pv
        pv = jnp.where(m, dc, pv)
        pc = jnp.where(m, c, pc)

    lv = jnp.min(pv, axis=1, keepdims=True)
    lane = jax.lax.broadcasted_iota(jnp.int32, pv.shape, 1)
    cand = pc * 128 + lane
    li = jnp.min(jnp.where(pv == lv, cand, N_EMB), axis=1, keepdims=True)
    li = li + j * BK

    @pl.when(j == 0)
    def _init():
        bestv_ref[...] = lv
        besti_ref[...] = li

    @pl.when(j > 0)
    def _update():
        upd = lv < bestv_ref[...]
        bestv_ref[...] = jnp.where(upd, lv, bestv_ref[...])
        besti_ref[...] = jnp.where(upd, li, besti_ref[...])

    @pl.when(j == nkt - 1)
    def _emit():
        idx_ref[...] = besti_ref[...]


@functools.partial(jax.jit, static_argnames=())
def _vq_indices(z2, W, a, b):
    grid = (N_TOK // BN, N_EMB // BK)
    idx = pl.pallas_call(
        _vq_body,
        grid=grid,
        in_specs=[
            pl.BlockSpec((BN, 1), lambda i, j: (i, 0)),
            pl.BlockSpec((1, BK), lambda i, j: (0, j)),
            pl.BlockSpec((BN, DIM), lambda i, j: (i, 0)),
            pl.BlockSpec((N_EMB, DIM), lambda i, j: (0, 0)),
        ],
        out_specs=pl.BlockSpec((BN, 1), lambda i, j: (i, 0)),
        out_shape=jax.ShapeDtypeStruct((N_TOK, 1), jnp.int32),
        scratch_shapes=[
            pltpu.VMEM((BN, 1), jnp.float32),
            pltpu.VMEM((BN, 1), jnp.int32),
        ],
        compiler_params=pltpu.CompilerParams(
            dimension_semantics=("parallel", "arbitrary"),
        ),
    )(a, b, z2, W)
    return idx.reshape(N_TOK)


def kernel(z, W):
    a = jnp.sum(z ** 2, axis=1, keepdims=True)
    b = jnp.sum(W ** 2, axis=1).reshape(1, N_EMB)
    indices = _vq_indices(z + z, W, a, b)
    z_q = z
    return (z_q, indices)


# speedup vs baseline: 1.5850x; 1.0004x over previous
import functools

import jax
import jax.numpy as jnp
from jax.experimental import pallas as pl
from jax.experimental.pallas import tpu as pltpu

N_TOK = 32768
N_EMB = 8192
DIM = 256

BN = 512
BK = 2048


def _vq_body(a_ref, b_ref, z_ref, w_ref, idx_ref, bestv_ref, besti_ref):
    j = pl.program_id(1)
    nkt = pl.num_programs(1)

    w = w_ref[pl.ds(j * BK, BK), :]
    s2 = jax.lax.dot_general(
        z_ref[...], w,
        dimension_numbers=(((1,), (1,)), ((), ())),
        preferred_element_type=jnp.float32,
    )
    d = (a_ref[...] + b_ref[...]) - s2

    nch = BK // 128
    pv = d[:, 0:128]
    pc = jnp.zeros(pv.shape, jnp.int32)
    for c in range(1, nch):
        dc = d[:, c * 128:(c + 1) * 128]
        m = dc < pv
        pv = jnp.where(m, dc, pv)
        pc = jnp.where(m, c, pc)

    lv = jnp.min(pv, axis=1, keepdims=True)
    lane = jax.lax.broadcasted_iota(jnp.int32, pv.shape, 1)
    cand = pc * 128 + lane
    li = jnp.min(jnp.where(pv == lv, cand, N_EMB), axis=1, keepdims=True)
    li = li + j * BK

    @pl.when(j == 0)
    def _init():
        bestv_ref[...] = lv
        besti_ref[...] = li

    @pl.when(j > 0)
    def _update():
        upd = lv < bestv_ref[...]
        bestv_ref[...] = jnp.where(upd, lv, bestv_ref[...])
        besti_ref[...] = jnp.where(upd, li, besti_ref[...])

    @pl.when(j == nkt - 1)
    def _emit():
        idx_ref[...] = besti_ref[...]


@functools.partial(jax.jit, static_argnames=())
def _vq_indices(z2, W, a, b):
    grid = (N_TOK // BN, N_EMB // BK)
    idx = pl.pallas_call(
        _vq_body,
        grid=grid,
        in_specs=[
            pl.BlockSpec((BN, 1), lambda i, j: (i, 0)),
            pl.BlockSpec((1, BK), lambda i, j: (0, j)),
            pl.BlockSpec((BN, DIM), lambda i, j: (i, 0)),
            pl.BlockSpec((N_EMB, DIM), lambda i, j: (0, 0)),
        ],
        out_specs=pl.BlockSpec((BN, 1), lambda i, j: (i, 0)),
        out_shape=jax.ShapeDtypeStruct((N_TOK, 1), jnp.int32),
        scratch_shapes=[
            pltpu.VMEM((BN, 1), jnp.float32),
            pltpu.VMEM((BN, 1), jnp.int32),
        ],
        compiler_params=pltpu.CompilerParams(
            dimension_semantics=("parallel", "arbitrary"),
        ),
    )(a, b, z2, W)
    return idx.reshape(N_TOK)


def kernel(z, W):
    a = jnp.sum(z ** 2, axis=1, keepdims=True)
    b = jnp.sum(W ** 2, axis=1).reshape(1, N_EMB)
    indices = _vq_indices((z + z).astype(jnp.bfloat16), W.astype(jnp.bfloat16), a, b)
    z_q = z
    return (z_q, indices)


# speedup vs baseline: 2.3590x; 1.4883x over previous
import functools

import jax
import jax.numpy as jnp
from jax.experimental import pallas as pl
from jax.experimental.pallas import tpu as pltpu

N_TOK = 32768
N_EMB = 8192
DIM = 256

BN = 512
SB = 512


def _vq_body(a_ref, b_ref, z_ref, w_ref, idx_ref, zq_ref):
    a = a_ref[...]
    zf = z_ref[...]
    zq_ref[...] = zf
    z = (zf + zf).astype(jnp.bfloat16)
    bestv = None
    for t in range(N_EMB // SB):
        w = w_ref[pl.ds(t * SB, SB), :]
        s2 = jax.lax.dot_general(
            z, w,
            dimension_numbers=(((1,), (1,)), ((), ())),
            preferred_element_type=jnp.float32,
        )
        d = (a + b_ref[:, t * SB:(t + 1) * SB]) - s2

        base = t * (SB // 128)
        for c in range(SB // 128):
            dc = d[:, c * 128:(c + 1) * 128]
            if bestv is None:
                bestv, bestc = dc, jnp.zeros(dc.shape, jnp.int32)
            else:
                m = dc < bestv
                bestv = jnp.where(m, dc, bestv)
                bestc = jnp.where(m, base + c, bestc)

    lv = jnp.min(bestv, axis=1, keepdims=True)
    lane = jax.lax.broadcasted_iota(jnp.int32, bestv.shape, 1)
    cand = bestc * 128 + lane
    idx_ref[...] = jnp.min(jnp.where(bestv == lv, cand, N_EMB),
                           axis=1, keepdims=True)


@functools.partial(jax.jit, static_argnames=())
def _vq_fused(z, Wb, a, b):
    idx, z_q = pl.pallas_call(
        _vq_body,
        grid=(N_TOK // BN,),
        in_specs=[
            pl.BlockSpec((BN, 1), lambda i: (i, 0)),
            pl.BlockSpec((1, N_EMB), lambda i: (0, 0)),
            pl.BlockSpec((BN, DIM), lambda i: (i, 0)),
            pl.BlockSpec((N_EMB, DIM), lambda i: (0, 0)),
        ],
        out_specs=[
            pl.BlockSpec((BN, 1), lambda i: (i, 0)),
            pl.BlockSpec((BN, DIM), lambda i: (i, 0)),
        ],
        out_shape=[
            jax.ShapeDtypeStruct((N_TOK, 1), jnp.int32),
            jax.ShapeDtypeStruct((N_TOK, DIM), jnp.float32),
        ],
        compiler_params=pltpu.CompilerParams(
            dimension_semantics=("parallel",),
        ),
    )(a, b, z, Wb)
    return idx.reshape(N_TOK), z_q


def kernel(z, W):
    a = jnp.sum(z ** 2, axis=1, keepdims=True)
    b = jnp.sum(W ** 2, axis=1).reshape(1, N_EMB)
    indices, z_q = _vq_fused(z, W.astype(jnp.bfloat16), a, b)
    return (z_q, indices)


# speedup vs baseline: 2.5177x; 1.0673x over previous
import functools

import jax
import jax.numpy as jnp
from jax.experimental import pallas as pl
from jax.experimental.pallas import tpu as pltpu

N_TOK = 32768
N_EMB = 8192
DIM = 256

BN = 512
SB = 512


def _vq_body(b_ref, z_ref, w_ref, idx_ref, zq_ref):
    zf = z_ref[...]
    zq_ref[...] = zf
    a = jnp.sum(zf ** 2, axis=1, keepdims=True)
    z = (zf + zf).astype(jnp.bfloat16)
    bestv = None
    for t in range(N_EMB // SB):
        w = w_ref[pl.ds(t * SB, SB), :]
        s2 = jax.lax.dot_general(
            z, w,
            dimension_numbers=(((1,), (1,)), ((), ())),
            preferred_element_type=jnp.float32,
        )
        d = (a + b_ref[:, t * SB:(t + 1) * SB]) - s2

        base = t * (SB // 128)
        for c in range(SB // 128):
            dc = d[:, c * 128:(c + 1) * 128]
            if bestv is None:
                bestv, bestc = dc, jnp.zeros(dc.shape, jnp.int32)
            else:
                m = dc < bestv
                bestv = jnp.where(m, dc, bestv)
                bestc = jnp.where(m, base + c, bestc)

    lv = jnp.min(bestv, axis=1, keepdims=True)
    lane = jax.lax.broadcasted_iota(jnp.int32, bestv.shape, 1)
    cand = bestc * 128 + lane
    idx_ref[...] = jnp.min(jnp.where(bestv == lv, cand, N_EMB),
                           axis=1, keepdims=True)


@functools.partial(jax.jit, static_argnames=())
def _vq_fused(z, Wb, b):
    idx, z_q = pl.pallas_call(
        _vq_body,
        grid=(N_TOK // BN,),
        in_specs=[
            pl.BlockSpec((1, N_EMB), lambda i: (0, 0)),
            pl.BlockSpec((BN, DIM), lambda i: (i, 0)),
            pl.BlockSpec((N_EMB, DIM), lambda i: (0, 0)),
        ],
        out_specs=[
            pl.BlockSpec((BN, 1), lambda i: (i, 0)),
            pl.BlockSpec((BN, DIM), lambda i: (i, 0)),
        ],
        out_shape=[
            jax.ShapeDtypeStruct((N_TOK, 1), jnp.int32),
            jax.ShapeDtypeStruct((N_TOK, DIM), jnp.float32),
        ],
        compiler_params=pltpu.CompilerParams(
            dimension_semantics=("parallel",),
        ),
    )(b, z, Wb)
    return idx.reshape(N_TOK), z_q


def kernel(z, W):
    b = jnp.sum(W ** 2, axis=1).reshape(1, N_EMB)
    indices, z_q = _vq_fused(z, W.astype(jnp.bfloat16), b)
    return (z_q, indices)


# speedup vs baseline: 2.7067x; 1.0751x over previous
import functools

import jax
import jax.numpy as jnp
from jax.experimental import pallas as pl
from jax.experimental.pallas import tpu as pltpu

N_TOK = 32768
N_EMB = 8192
DIM = 256

BN = 1024
SB = 512


def _vq_body(b_ref, z_ref, w_ref, idx_ref, zq_ref):
    zf = z_ref[...]
    zq_ref[...] = zf
    a = jnp.sum(zf ** 2, axis=1, keepdims=True)
    z = (zf + zf).astype(jnp.bfloat16)
    bestv = None
    for t in range(N_EMB // SB):
        w = w_ref[pl.ds(t * SB, SB), :]
        s2 = jax.lax.dot_general(
            z, w,
            dimension_numbers=(((1,), (1,)), ((), ())),
            preferred_element_type=jnp.float32,
        )
        d = (a + b_ref[:, t * SB:(t + 1) * SB]) - s2

        base = t * (SB // 128)
        for c in range(SB // 128):
            dc = d[:, c * 128:(c + 1) * 128]
            if bestv is None:
                bestv, bestc = dc, jnp.zeros(dc.shape, jnp.int32)
            else:
                m = dc < bestv
                bestv = jnp.where(m, dc, bestv)
                bestc = jnp.where(m, base + c, bestc)

    lv = jnp.min(bestv, axis=1, keepdims=True)
    lane = jax.lax.broadcasted_iota(jnp.int32, bestv.shape, 1)
    cand = bestc * 128 + lane
    idx_ref[...] = jnp.min(jnp.where(bestv == lv, cand, N_EMB),
                           axis=1, keepdims=True)


@functools.partial(jax.jit, static_argnames=())
def _vq_fused(z, Wb, b):
    idx, z_q = pl.pallas_call(
        _vq_body,
        grid=(N_TOK // BN,),
        in_specs=[
            pl.BlockSpec((1, N_EMB), lambda i: (0, 0)),
            pl.BlockSpec((BN, DIM), lambda i: (i, 0)),
            pl.BlockSpec((N_EMB, DIM), lambda i: (0, 0)),
        ],
        out_specs=[
            pl.BlockSpec((BN, 1), lambda i: (i, 0)),
            pl.BlockSpec((BN, DIM), lambda i: (i, 0)),
        ],
        out_shape=[
            jax.ShapeDtypeStruct((N_TOK, 1), jnp.int32),
            jax.ShapeDtypeStruct((N_TOK, DIM), jnp.float32),
        ],
        compiler_params=pltpu.CompilerParams(
            dimension_semantics=("parallel",),
        ),
    )(b, z, Wb)
    return idx.reshape(N_TOK), z_q


def kernel(z, W):
    b = jnp.sum(W ** 2, axis=1).reshape(1, N_EMB)
    indices, z_q = _vq_fused(z, W.astype(jnp.bfloat16), b)
    return (z_q, indices)
